# baseline (speedup 1.0000x reference)
import functools

import jax
import jax.numpy as jnp
from jax.experimental import pallas as pl
from jax.experimental.pallas import tpu as pltpu
from jax.experimental.pallas import tpu_sc as plsc


_BLK = 64
_BLK_LOG2 = 6
_EPB = 1


def _routing_body(x_ref, gw_ref, pos_ref, meta_ref):
    T, D = x_ref.shape
    E = gw_ref.shape[1]
    NUPAD = 128
    logits = jax.lax.dot_general(
        x_ref[...].astype(jnp.bfloat16), gw_ref[...].astype(jnp.bfloat16),
        (((1,), (0,)), ((), ())),
        preferred_element_type=jnp.float32,
    )
    iota_e = jax.lax.broadcasted_iota(jnp.int32, (T, E), 1)
    mx = jnp.max(logits, axis=1, keepdims=True)
    eid = jnp.min(jnp.where(logits == mx, iota_e, E), axis=1, keepdims=True)
    onehot = (iota_e == eid).astype(jnp.int32)

    incl = onehot
    d = 1
    while d < T:
        shifted = jnp.concatenate(
            [jnp.zeros((d, E), jnp.int32), incl[: T - d, :]], axis=0)
        incl = incl + shifted
        d *= 2
    rank_excl = incl - onehot
    counts = incl[T - 1:T, :]

    cincl = counts
    d = 1
    while d < E:
        shifted = jnp.concatenate(
            [jnp.zeros((1, d), jnp.int32), cincl[:, : E - d]], axis=1)
        cincl = cincl + shifted
        d *= 2
    off_excl = cincl - counts

    pos = jnp.sum(onehot * (rank_excl + off_excl), axis=1, keepdims=True)
    pos_ref[...] = pos

    counts_c = jax.lax.dot_general(
        onehot.astype(jnp.float32), jnp.ones((T, 1), jnp.float32),
        (((0,), (0,)), ((), ())),
        preferred_element_type=jnp.float32).astype(jnp.int32)
    cin = counts_c
    d = 1
    while d < E:
        cin = cin + jnp.concatenate(
            [jnp.zeros((d, 1), jnp.int32), cin[: E - d, :]], axis=0)
        d *= 2
    off_lo_c = cin - counts_c
    off_hi_c = cin
    first_tile_c = jax.lax.shift_right_arithmetic(off_lo_c, _BLK_LOG2)
    last_tile_c = jax.lax.shift_right_arithmetic(off_hi_c - 1, _BLK_LOG2)
    upe_c = jnp.where(counts_c > 0, last_tile_c - first_tile_c + 1, 0)
    uin = upe_c
    d = 1
    while d < E:
        uin = uin + jnp.concatenate(
            [jnp.zeros((d, 1), jnp.int32), uin[: E - d, :]], axis=0)
        d *= 2
    ustart_c = uin - upe_c
    total = jnp.sum(uin[E - 1:E, :], axis=1, keepdims=True)

    u_row = jax.lax.broadcasted_iota(jnp.int32, (1, NUPAD), 1)
    uc = jnp.minimum(u_row, total - 1)
    in_e = (uc >= ustart_c) & (uc < ustart_c + upe_c)
    e_ids_c = jax.lax.broadcasted_iota(jnp.int32, (E, 1), 0)
    zero = jnp.zeros((E, NUPAD), jnp.int32)
    e_arr = jnp.sum(jnp.where(in_e, e_ids_c + zero, 0), axis=0, keepdims=True)
    m_arr = jnp.sum(
        jnp.where(in_e, first_tile_c + (uc - ustart_c), 0), axis=0,
        keepdims=True)
    valid_u = u_row < total
    start_arr = jnp.where(
        valid_u,
        jnp.sum(jnp.where(in_e, off_lo_c + zero, 0), axis=0, keepdims=True), 0)
    end_arr = jnp.where(
        valid_u,
        jnp.sum(jnp.where(in_e, off_hi_c + zero, 0), axis=0, keepdims=True), 0)
    prev_m = jnp.concatenate([m_arr[:, :1], m_arr[:, : NUPAD - 1]], axis=1)
    first_arr = ((u_row == 0) | (m_arr != prev_m)).astype(jnp.int32)

    meta_ref[...] = jnp.concatenate(
        [e_arr, m_arr, first_arr, start_arr, end_arr,
         jnp.zeros((3, NUPAD), jnp.int32)], axis=0)


def _route(x_flat, gate_w):
    T, D = x_flat.shape
    return pl.pallas_call(
        _routing_body,
        out_shape=(
            jax.ShapeDtypeStruct((T, 1), jnp.int32),
            jax.ShapeDtypeStruct((8, 128), jnp.int32),
        ),
    )(x_flat, gate_w)


_SC_NW = 32


def _sc_scatter_rows(rows, idx):
    T, D = rows.shape
    per_w = T // _SC_NW
    mesh = plsc.VectorSubcoreMesh(core_axis_name="c", subcore_axis_name="s")

    @functools.partial(
        pl.kernel,
        out_type=jax.ShapeDtypeStruct((T, D), rows.dtype),
        mesh=mesh,
        scratch_types=[
            pltpu.VMEM((per_w,), jnp.int32),
            pltpu.VMEM((per_w, D), rows.dtype),
            pltpu.SemaphoreType.DMA,
        ],
    )
    def _k(x_hbm, i_hbm, o_hbm, idx_v, rows_v, sem):
        wid = jax.lax.axis_index("s") * 2 + jax.lax.axis_index("c")
        base = wid * per_w
        pltpu.sync_copy(i_hbm.at[pl.ds(base, per_w)], idx_v)
        pltpu.sync_copy(x_hbm.at[pl.ds(base, per_w)], rows_v)
        pltpu.async_copy(rows_v, o_hbm.at[idx_v], sem).wait()

    return _k(rows, idx)


def _sc_gather_rows(table, idx):
    T = idx.shape[0]
    D = table.shape[1]
    per_w = T // _SC_NW
    mesh = plsc.VectorSubcoreMesh(core_axis_name="c", subcore_axis_name="s")

    @functools.partial(
        pl.kernel,
        out_type=jax.ShapeDtypeStruct((T, D), table.dtype),
        mesh=mesh,
        scratch_types=[
            pltpu.VMEM((per_w,), jnp.int32),
            pltpu.VMEM((per_w, D), table.dtype),
            pltpu.SemaphoreType.DMA,
        ],
    )
    def _k(t_hbm, i_hbm, o_hbm, idx_v, rows_v, sem):
        wid = jax.lax.axis_index("s") * 2 + jax.lax.axis_index("c")
        base = wid * per_w
        pltpu.sync_copy(i_hbm.at[pl.ds(base, per_w)], idx_v)
        pltpu.async_copy(t_hbm.at[idx_v], rows_v, sem).wait()
        pltpu.sync_copy(rows_v, o_hbm.at[pl.ds(base, per_w)])

    return _k(table, idx)


def _ffn_body(meta_ref, xs_ref, wg_ref, wu_ref, wd_ref, out_ref):
    u = pl.program_id(0)
    BLK = xs_ref.shape[0]
    xb = xs_ref[...].astype(jnp.bfloat16)
    wg = wg_ref[0].astype(jnp.bfloat16)
    wu = wu_ref[0].astype(jnp.bfloat16)
    wd = wd_ref[0].astype(jnp.bfloat16)
    g = jnp.dot(xb, wg, preferred_element_type=jnp.float32)
    v = jnp.dot(xb, wu, preferred_element_type=jnp.float32)
    h = (g * jax.nn.sigmoid(g) * v).astype(jnp.bfloat16)
    y = jnp.dot(h, wd, preferred_element_type=jnp.float32)

    rows = meta_ref[1, u] * BLK + jax.lax.broadcasted_iota(
        jnp.int32, (BLK, 1), 0)
    mask = (rows >= meta_ref[3, u]) & (rows < meta_ref[4, u])
    y = jnp.where(mask, y, 0.0)

    @pl.when(meta_ref[2, u] == 1)
    def _init():
        out_ref[...] = y

    @pl.when(meta_ref[2, u] == 0)
    def _acc():
        out_ref[...] = out_ref[...] + y


def _ffn(xs, Wg, Wu, Wd, meta):
    T, D = xs.shape
    E, _, H = Wg.shape
    NU = T // _BLK + E
    grid_spec = pltpu.PrefetchScalarGridSpec(
        num_scalar_prefetch=1,
        grid=(NU,),
        in_specs=[
            pl.BlockSpec((_BLK, D), lambda u, meta: (meta[1, u], 0)),
            pl.BlockSpec((_EPB, D, H), lambda u, meta: (meta[0, u] // _EPB, 0, 0)),
            pl.BlockSpec((_EPB, D, H), lambda u, meta: (meta[0, u] // _EPB, 0, 0)),
            pl.BlockSpec((_EPB, H, D), lambda u, meta: (meta[0, u] // _EPB, 0, 0)),
        ],
        out_specs=pl.BlockSpec((_BLK, D), lambda u, meta: (meta[1, u], 0)),
    )
    return pl.pallas_call(
        _ffn_body,
        grid_spec=grid_spec,
        out_shape=jax.ShapeDtypeStruct((T, D), jnp.float32),
    )(meta, xs, Wg, Wu, Wd)


def kernel(x, gate_w, Wg, Wu, Wd):
    Bb, Tt, D = x.shape
    E = gate_w.shape[1]
    x_flat = x.reshape(Bb * Tt, D)
    T = Bb * Tt

    pos2d, meta = _route(x_flat, gate_w)
    pos = pos2d.reshape(T)

    xs = _sc_scatter_rows(x_flat, pos)
    ys = _ffn(xs, Wg, Wu, Wd, meta)
    out = _sc_gather_rows(ys, pos)
    return out.reshape(Bb, Tt, D)


# speedup vs baseline: 1.1105x; 1.1105x over previous
import functools

import jax
import jax.numpy as jnp
from jax.experimental import pallas as pl
from jax.experimental.pallas import tpu as pltpu
from jax.experimental.pallas import tpu_sc as plsc


_BLK = 128
_BLK_LOG2 = 7


def _routing_body(x_ref, gw_ref, pos_ref, meta_ref):
    T, D = x_ref.shape
    E = gw_ref.shape[1]
    NUPAD = 128
    logits = jax.lax.dot_general(
        x_ref[...].astype(jnp.bfloat16), gw_ref[...].astype(jnp.bfloat16),
        (((1,), (0,)), ((), ())),
        preferred_element_type=jnp.float32,
    )
    iota_e = jax.lax.broadcasted_iota(jnp.int32, (T, E), 1)
    mx = jnp.max(logits, axis=1, keepdims=True)
    eid = jnp.min(jnp.where(logits == mx, iota_e, E), axis=1, keepdims=True)
    onehot = (iota_e == eid).astype(jnp.int32)

    incl = onehot
    d = 1
    while d < T:
        shifted = jnp.concatenate(
            [jnp.zeros((d, E), jnp.int32), incl[: T - d, :]], axis=0)
        incl = incl + shifted
        d *= 2
    rank_excl = incl - onehot
    counts = incl[T - 1:T, :]

    cincl = counts
    d = 1
    while d < E:
        shifted = jnp.concatenate(
            [jnp.zeros((1, d), jnp.int32), cincl[:, : E - d]], axis=1)
        cincl = cincl + shifted
        d *= 2
    off_excl = cincl - counts

    pos = jnp.sum(onehot * (rank_excl + off_excl), axis=1, keepdims=True)
    pos_ref[...] = pos

    counts_c = jax.lax.dot_general(
        onehot.astype(jnp.float32), jnp.ones((T, 1), jnp.float32),
        (((0,), (0,)), ((), ())),
        preferred_element_type=jnp.float32).astype(jnp.int32)
    cin = counts_c
    d = 1
    while d < E:
        cin = cin + jnp.concatenate(
            [jnp.zeros((d, 1), jnp.int32), cin[: E - d, :]], axis=0)
        d *= 2
    off_lo_c = cin - counts_c
    off_hi_c = cin
    first_tile_c = jax.lax.shift_right_arithmetic(off_lo_c, _BLK_LOG2)
    last_tile_c = jax.lax.shift_right_arithmetic(off_hi_c - 1, _BLK_LOG2)
    upe_c = jnp.where(counts_c > 0, last_tile_c - first_tile_c + 1, 0)
    uin = upe_c
    d = 1
    while d < E:
        uin = uin + jnp.concatenate(
            [jnp.zeros((d, 1), jnp.int32), uin[: E - d, :]], axis=0)
        d *= 2
    ustart_c = uin - upe_c
    total = jnp.sum(uin[E - 1:E, :], axis=1, keepdims=True)

    u_row = jax.lax.broadcasted_iota(jnp.int32, (1, NUPAD), 1)
    uc = jnp.minimum(u_row, total - 1)
    in_e = (uc >= ustart_c) & (uc < ustart_c + upe_c)
    e_ids_c = jax.lax.broadcasted_iota(jnp.int32, (E, 1), 0)
    zero = jnp.zeros((E, NUPAD), jnp.int32)
    e_arr = jnp.sum(jnp.where(in_e, e_ids_c + zero, 0), axis=0, keepdims=True)
    m_arr = jnp.sum(
        jnp.where(in_e, first_tile_c + (uc - ustart_c), 0), axis=0,
        keepdims=True)
    valid_u = u_row < total
    start_arr = jnp.where(
        valid_u,
        jnp.sum(jnp.where(in_e, off_lo_c + zero, 0), axis=0, keepdims=True), 0)
    end_arr = jnp.where(
        valid_u,
        jnp.sum(jnp.where(in_e, off_hi_c + zero, 0), axis=0, keepdims=True), 0)
    prev_m = jnp.concatenate([m_arr[:, :1], m_arr[:, : NUPAD - 1]], axis=1)
    first_arr = ((u_row == 0) | (m_arr != prev_m)).astype(jnp.int32)

    meta_ref[...] = jnp.concatenate(
        [e_arr, m_arr, first_arr, start_arr, end_arr,
         jnp.zeros((3, NUPAD), jnp.int32)], axis=0)


def _route(x_flat, gate_w):
    T, D = x_flat.shape
    return pl.pallas_call(
        _routing_body,
        out_shape=(
            jax.ShapeDtypeStruct((T, 1), jnp.int32),
            jax.ShapeDtypeStruct((8, 128), jnp.int32),
        ),
    )(x_flat, gate_w)


_SC_NW = 32


def _sc_scatter_rows(rows, idx):
    T, D = rows.shape
    per_w = T // _SC_NW
    mesh = plsc.VectorSubcoreMesh(core_axis_name="c", subcore_axis_name="s")

    @functools.partial(
        pl.kernel,
        out_type=jax.ShapeDtypeStruct((T, D), rows.dtype),
        mesh=mesh,
        scratch_types=[
            pltpu.VMEM((per_w,), jnp.int32),
            pltpu.VMEM((per_w, D), rows.dtype),
            pltpu.SemaphoreType.DMA,
        ],
    )
    def _k(x_hbm, i_hbm, o_hbm, idx_v, rows_v, sem):
        wid = jax.lax.axis_index("s") * 2 + jax.lax.axis_index("c")
        base = wid * per_w
        pltpu.sync_copy(i_hbm.at[pl.ds(base, per_w)], idx_v)
        pltpu.sync_copy(x_hbm.at[pl.ds(base, per_w)], rows_v)
        pltpu.async_copy(rows_v, o_hbm.at[idx_v], sem).wait()

    return _k(rows, idx)


def _sc_gather_rows(table, idx):
    T = idx.shape[0]
    D = table.shape[1]
    per_w = T // _SC_NW
    mesh = plsc.VectorSubcoreMesh(core_axis_name="c", subcore_axis_name="s")

    @functools.partial(
        pl.kernel,
        out_type=jax.ShapeDtypeStruct((T, D), table.dtype),
        mesh=mesh,
        scratch_types=[
            pltpu.VMEM((per_w,), jnp.int32),
            pltpu.VMEM((per_w, D), table.dtype),
            pltpu.SemaphoreType.DMA,
        ],
    )
    def _k(t_hbm, i_hbm, o_hbm, idx_v, rows_v, sem):
        wid = jax.lax.axis_index("s") * 2 + jax.lax.axis_index("c")
        base = wid * per_w
        pltpu.sync_copy(i_hbm.at[pl.ds(base, per_w)], idx_v)
        pltpu.async_copy(t_hbm.at[idx_v], rows_v, sem).wait()
        pltpu.sync_copy(rows_v, o_hbm.at[pl.ds(base, per_w)])

    return _k(table, idx)


def _ffn_body(meta_ref, xs_ref, wg_ref, wu_ref, wd_ref, out_ref):
    u = pl.program_id(0)
    BLK = xs_ref.shape[0]
    xb = xs_ref[...].astype(jnp.bfloat16)
    wg = wg_ref[0].astype(jnp.bfloat16)
    wu = wu_ref[0].astype(jnp.bfloat16)
    wd = wd_ref[0].astype(jnp.bfloat16)
    g = jnp.dot(xb, wg, preferred_element_type=jnp.float32)
    v = jnp.dot(xb, wu, preferred_element_type=jnp.float32)
    h = (g * jax.nn.sigmoid(g) * v).astype(jnp.bfloat16)
    y = jnp.dot(h, wd, preferred_element_type=jnp.float32)

    rows = meta_ref[1, u] * BLK + jax.lax.broadcasted_iota(
        jnp.int32, (BLK, 1), 0)
    mask = (rows >= meta_ref[3, u]) & (rows < meta_ref[4, u])
    y = jnp.where(mask, y, 0.0)

    @pl.when(meta_ref[2, u] == 1)
    def _init():
        out_ref[...] = y

    @pl.when(meta_ref[2, u] == 0)
    def _acc():
        out_ref[...] = out_ref[...] + y


def _ffn(xs, Wg, Wu, Wd, meta):
    T, D = xs.shape
    E, _, H = Wg.shape
    NU = T // _BLK + E
    grid_spec = pltpu.PrefetchScalarGridSpec(
        num_scalar_prefetch=1,
        grid=(NU,),
        in_specs=[
            pl.BlockSpec((_BLK, D), lambda u, meta: (meta[1, u], 0)),
            pl.BlockSpec((1, D, H), lambda u, meta: (meta[0, u], 0, 0)),
            pl.BlockSpec((1, D, H), lambda u, meta: (meta[0, u], 0, 0)),
            pl.BlockSpec((1, H, D), lambda u, meta: (meta[0, u], 0, 0)),
        ],
        out_specs=pl.BlockSpec((_BLK, D), lambda u, meta: (meta[1, u], 0)),
    )
    return pl.pallas_call(
        _ffn_body,
        grid_spec=grid_spec,
        out_shape=jax.ShapeDtypeStruct((T, D), jnp.float32),
    )(meta, xs, Wg, Wu, Wd)


def kernel(x, gate_w, Wg, Wu, Wd):
    Bb, Tt, D = x.shape
    E = gate_w.shape[1]
    x_flat = x.reshape(Bb * Tt, D)
    T = Bb * Tt

    pos2d, meta = _route(x_flat, gate_w)
    pos = pos2d.reshape(T)

    xs = _sc_scatter_rows(x_flat, pos)
    ys = _ffn(xs, Wg, Wu, Wd, meta)
    out = _sc_gather_rows(ys, pos)
    return out.reshape(Bb, Tt, D)
